# baseline (speedup 1.0000x reference)
import functools

import jax
import jax.numpy as jnp
from jax.experimental import pallas as pl


def _mbox_kernel(x_loc_ref, x_conf_ref, y_ref, loc_ref, conf_ref, np_ref):
    b = pl.program_id(0)

    @pl.when(b == 0)
    def _init():
        loc_ref[...] = jnp.zeros_like(loc_ref)
        conf_ref[...] = jnp.zeros_like(conf_ref)
        np_ref[...] = jnp.zeros_like(np_ref)

    xc = x_conf_ref[0]
    yv = y_ref[0]
    xl = x_loc_ref[0]
    P, C = xc.shape

    labels = yv[:, 0].astype(jnp.int32)
    pos = labels > 0
    num_pos = jnp.sum(pos.astype(jnp.int32))

    d = xl - yv[:, 1:]
    ad = jnp.abs(d)
    sl1 = jnp.where(ad < 1.0, 0.5 * d * d, ad - 0.5)
    loc_row = jnp.sum(jnp.where(pos[:, None], sl1, 0.0))

    m = jnp.max(xc, axis=1)
    e = jnp.exp(xc - m[:, None])
    s = jnp.sum(e, axis=1)
    lse = jnp.log(s) + m
    lane = jax.lax.broadcasted_iota(jnp.int32, (P, C), 1)
    gathered = jnp.sum(jnp.where(lane == labels[:, None], xc, 0.0), axis=1)
    ce = lse - gathered
    l = jnp.where(pos, 0.0, ce)

    k = jnp.minimum(3 * num_pos, P - 1)

    bits = jax.lax.bitcast_convert_type(l, jnp.int32)

    def body(_, carry):
        lo, hi = carry
        mid = lo + (hi - lo + 1) // 2
        cnt = jnp.sum((bits >= mid).astype(jnp.int32))
        ok = cnt >= k
        return jnp.where(ok, mid, lo), jnp.where(ok, hi, mid - 1)

    t_bits, _ = jax.lax.fori_loop(
        0, 31, body, (jnp.int32(0), jnp.int32(0x7F800000))
    )
    t = jax.lax.bitcast_convert_type(t_bits, jnp.float32)

    gt = bits > t_bits
    cnt_gt = jnp.sum(gt.astype(jnp.int32))
    sum_gt = jnp.sum(jnp.where(gt, l, 0.0))
    neg_sum = sum_gt + (k - cnt_gt).astype(jnp.float32) * t
    pos_sum = jnp.sum(jnp.where(pos, ce, 0.0))
    conf_row = pos_sum + jnp.where(k > 0, neg_sum, 0.0)

    loc_ref[...] += jnp.full(loc_ref.shape, loc_row, jnp.float32)
    conf_ref[...] += jnp.full(conf_ref.shape, conf_row, jnp.float32)
    np_ref[...] += jnp.full(np_ref.shape, num_pos.astype(jnp.float32), jnp.float32)


@jax.jit
def kernel(x_loc, x_conf, y):
    B, P, C = x_conf.shape
    out_shape = jax.ShapeDtypeStruct((8, 128), jnp.float32)
    acc_spec = pl.BlockSpec((8, 128), lambda b: (0, 0))
    loc_s, conf_s, np_s = pl.pallas_call(
        _mbox_kernel,
        grid=(B,),
        in_specs=[
            pl.BlockSpec((1, P, 4), lambda b: (b, 0, 0)),
            pl.BlockSpec((1, P, C), lambda b: (b, 0, 0)),
            pl.BlockSpec((1, P, 5), lambda b: (b, 0, 0)),
        ],
        out_specs=(acc_spec, acc_spec, acc_spec),
        out_shape=(out_shape, out_shape, out_shape),
    )(x_loc, x_conf, y)
    nf = np_s[0, 0]
    return (loc_s[0, 0] / nf, conf_s[0, 0] / nf)


# speedup vs baseline: 11.8810x; 11.8810x over previous
import jax
import jax.numpy as jnp
from jax.experimental import pallas as pl
from jax.experimental.pallas import tpu as pltpu


def _mbox_kernel(xc_ref, lab_ref, xl_ref, tgt_ref, loc_ref, conf_ref, np_ref,
                 bits_ref, k_ref):
    b = pl.program_id(0)
    nb = pl.num_programs(0)

    @pl.when(b == 0)
    def _init():
        loc_ref[...] = jnp.zeros_like(loc_ref)
        conf_ref[...] = jnp.zeros_like(conf_ref)
        np_ref[...] = jnp.zeros_like(np_ref)

    xc = xc_ref[0]
    C, P = xc.shape

    labels = lab_ref[0].astype(jnp.int32)
    pos = labels > 0
    num_pos = jnp.sum(pos.astype(jnp.int32))

    d = xl_ref[0] - tgt_ref[0]
    ad = jnp.abs(d)
    sl1 = jnp.where(ad < 1.0, 0.5 * d * d, ad - 0.5)
    loc_row = jnp.sum(jnp.where(pos, sl1, 0.0))

    m = jnp.max(xc, axis=0, keepdims=True)
    e = jnp.exp(xc - m)
    s = jnp.sum(e, axis=0, keepdims=True)
    lse = jnp.log(s) + m
    cidx = jax.lax.broadcasted_iota(jnp.int32, (C, P), 0)
    gathered = jnp.sum(jnp.where(cidx == labels, xc, 0.0), axis=0,
                       keepdims=True)
    ce = lse - gathered
    l = jnp.where(pos, 0.0, ce)

    pos_sum = jnp.sum(jnp.where(pos, ce, 0.0))

    bits_ref[pl.ds(b, 1), :] = jax.lax.bitcast_convert_type(l, jnp.int32)
    k = jnp.minimum(3 * num_pos, P - 1)
    k_ref[pl.ds(b, 1), :] = jnp.full((1, 128), k, jnp.int32)

    loc_ref[...] += jnp.full(loc_ref.shape, loc_row, jnp.float32)
    conf_ref[...] += jnp.full(conf_ref.shape, pos_sum, jnp.float32)
    np_ref[...] += jnp.full(np_ref.shape, num_pos.astype(jnp.float32),
                            jnp.float32)

    @pl.when(b == nb - 1)
    def _select():
        bits = bits_ref[...]
        kv = k_ref[:, 0:1]

        def body(_, carry):
            lo, hi = carry
            mid = lo + (hi - lo + 1) // 2
            cnt = jnp.sum((bits >= mid).astype(jnp.int32), axis=1,
                          keepdims=True)
            ok = cnt >= kv
            return jnp.where(ok, mid, lo), jnp.where(ok, hi, mid - 1)

        B = bits.shape[0]
        t_bits, _ = jax.lax.fori_loop(
            0, 31, body,
            (jnp.zeros((B, 1), jnp.int32),
             jnp.full((B, 1), 0x7F800000, jnp.int32)),
        )
        t = jax.lax.bitcast_convert_type(t_bits, jnp.float32)

        gt = bits > t_bits
        cnt_gt = jnp.sum(gt.astype(jnp.int32), axis=1, keepdims=True)
        l_all = jax.lax.bitcast_convert_type(bits, jnp.float32)
        sum_gt = jnp.sum(jnp.where(gt, l_all, 0.0), axis=1, keepdims=True)
        neg = sum_gt + (kv - cnt_gt).astype(jnp.float32) * t
        neg_total = jnp.sum(jnp.where(kv > 0, neg, 0.0))
        conf_ref[...] += jnp.full(conf_ref.shape, neg_total, jnp.float32)


@jax.jit
def kernel(x_loc, x_conf, y):
    B, P, C = x_conf.shape
    xc_t = jnp.swapaxes(x_conf, 1, 2)
    lab = y[:, :, 0].reshape(B, 1, P)
    tgt = jnp.swapaxes(y[:, :, 1:], 1, 2)
    xl_t = jnp.swapaxes(x_loc, 1, 2)

    out_shape = jax.ShapeDtypeStruct((8, 128), jnp.float32)
    acc_spec = pl.BlockSpec((8, 128), lambda b: (0, 0))
    loc_s, conf_s, np_s = pl.pallas_call(
        _mbox_kernel,
        grid=(B,),
        in_specs=[
            pl.BlockSpec((1, C, P), lambda b: (b, 0, 0)),
            pl.BlockSpec((1, 1, P), lambda b: (b, 0, 0)),
            pl.BlockSpec((1, 4, P), lambda b: (b, 0, 0)),
            pl.BlockSpec((1, 4, P), lambda b: (b, 0, 0)),
        ],
        out_specs=(acc_spec, acc_spec, acc_spec),
        out_shape=(out_shape, out_shape, out_shape),
        scratch_shapes=[
            pltpu.VMEM((B, P), jnp.int32),
            pltpu.VMEM((B, 128), jnp.int32),
        ],
    )(xc_t, lab, xl_t, tgt)
    nf = np_s[0, 0]
    return (loc_s[0, 0] / nf, conf_s[0, 0] / nf)
